# baseline (speedup 1.0000x reference)
import functools

import jax
import jax.numpy as jnp
from jax import lax
from jax.experimental import pallas as pl
from jax.experimental.pallas import tpu as pltpu
from jax.experimental.pallas import tpu_sc as plsc

_TABLE = 1000000
_D = 32
_H = 128
_NC, _NS = 2, 16
_NW = _NC * _NS
_CH = 512
_SUB = 128
_NSUB = _CH // _SUB

_HASH_C = (
    (387584, 8191),
    (242496, 104729),
    (935232, 97593),
)

_QP = 262144
_TBN = 2048
_LASTB = (_TABLE - 1) // _TBN


def _tc_transpose_packed(table_t):
    def body(t0, t1, t2, t3, o_r):
        eye = (lax.broadcasted_iota(jnp.int32, (_H, _H), 0)
               == lax.broadcasted_iota(jnp.int32, (_H, _H), 1)
               ).astype(jnp.float32)
        tall = jnp.concatenate([t[...] for t in (t0, t1, t2, t3)], axis=0)
        o_r[...] = lax.dot_general(
            tall, eye, (((0,), (0,)), ((), ())),
            preferred_element_type=jnp.float32)

    def in_spec(a):
        base = a * (_QP // _TBN)
        return pl.BlockSpec(
            (_D, _TBN),
            lambda i: (jnp.int32(0),
                       jnp.minimum(jnp.int32(base) + i, jnp.int32(_LASTB))))

    return pl.pallas_call(
        body,
        grid=(_QP // _TBN,),
        in_specs=[in_spec(a) for a in range(4)],
        out_specs=pl.BlockSpec((_TBN, 4 * _D), lambda i: (i, jnp.int32(0))),
        out_shape=jax.ShapeDtypeStruct((_QP, 4 * _D), jnp.float32),
    )(table_t, table_t, table_t, table_t)


_NBLK_FULL = _TABLE // 128
_TAIL0 = _NBLK_FULL * 128


def _sc_repack(table_t):
    mesh = plsc.VectorSubcoreMesh(core_axis_name="c", subcore_axis_name="s")
    base_per_w = _NBLK_FULL // _NW
    extra = _NBLK_FULL % _NW

    @functools.partial(
        pl.kernel,
        out_type=jax.ShapeDtypeStruct((_TABLE // 4, _H), jnp.float32),
        mesh=mesh,
        scratch_types=[
            pltpu.VMEM((_D, 128), jnp.float32),
            pltpu.VMEM((_D, _H), jnp.float32),
            pltpu.VMEM((_D, 64), jnp.float32),
            pltpu.VMEM((16, _H), jnp.float32),
        ],
        compiler_params=pltpu.CompilerParams(
            use_tc_tiling_on_sc=True, needs_layout_passes=False),
    )
    def k(tt_h, pk_h, slab_v, out_v, tslab_v, tout_v):
        wid = lax.axis_index("s") * _NC + lax.axis_index("c")
        nblk = jnp.int32(base_per_w) + (wid < extra).astype(jnp.int32)
        lanes = lax.broadcasted_iota(jnp.int32, (16,), 0)

        def blk(kk, carry):
            b = kk * jnp.int32(_NW) + wid
            pltpu.sync_copy(tt_h.at[:, pl.ds(b * jnp.int32(128), 128)],
                            slab_v)

            def row(r, rcarry):
                for j in range(4):
                    u = jnp.broadcast_to(r * jnp.int32(4) + j, (16,))
                    for g in range(2):
                        vals = plsc.load_gather(
                            slab_v, [lanes + jnp.int32(16 * g), u])
                        out_v[r, pl.ds(jnp.int32(32 * j + 16 * g), 16)] = vals
                return rcarry

            lax.fori_loop(jnp.int32(0), jnp.int32(_D), row, jnp.int32(0))
            pltpu.sync_copy(out_v, pk_h.at[pl.ds(b * jnp.int32(_D), _D)])
            return carry

        lax.fori_loop(jnp.int32(0), nblk, blk, jnp.int32(0))

        @pl.when(wid == jnp.int32(_NW - 1))
        def _():
            pltpu.sync_copy(tt_h.at[:, pl.ds(jnp.int32(_TAIL0), 64)],
                            tslab_v)

            def row(r, rcarry):
                for j in range(4):
                    u = jnp.broadcast_to(r * jnp.int32(4) + j, (16,))
                    for g in range(2):
                        vals = plsc.load_gather(
                            tslab_v, [lanes + jnp.int32(16 * g), u])
                        tout_v[r, pl.ds(jnp.int32(32 * j + 16 * g), 16)] = vals
                return rcarry

            lax.fori_loop(jnp.int32(0), jnp.int32(16), row, jnp.int32(0))
            pltpu.sync_copy(
                tout_v, pk_h.at[pl.ds(jnp.int32(_TAIL0 // 4), 16)])

    return k(table_t)


def _sc_gather_one(ids, prev, table_pk, n, c_hi, c_lo, remap):
    nchunks = n // _CH
    base_per_w = nchunks // _NW
    extra = nchunks % _NW
    mesh = plsc.VectorSubcoreMesh(core_axis_name="c", subcore_axis_name="s")

    @functools.partial(
        pl.kernel,
        out_type=jax.ShapeDtypeStruct((n, _D), jnp.float32),
        mesh=mesh,
        scratch_types=[
            pltpu.VMEM((_CH,), jnp.int32),
            pltpu.VMEM((_CH,), jnp.int32),
            pltpu.VMEM((_CH,), jnp.int32),
            pltpu.VMEM((_CH, _D), jnp.float32),
            pltpu.SemaphoreType.DMA,
        ],
        compiler_params=pltpu.CompilerParams(
            use_tc_tiling_on_sc=False, needs_layout_passes=False),
    )
    def k(ids_h, prev_h, t_h, e_h, ids_v, prev_v, idx_v, rows_v, sem):
        wid = lax.axis_index("s") * _NC + lax.axis_index("c")
        nch = jnp.int32(base_per_w) + (wid < extra).astype(jnp.int32)

        def chunk(c, carry):
            base = (c * jnp.int32(_NW) + wid) * jnp.int32(_CH)
            pltpu.sync_copy(ids_h.at[pl.ds(base, _CH)], ids_v)
            pltpu.sync_copy(prev_h.at[pl.ds(base, _CH)], prev_v)

            def hstep(i, hcarry):
                sl = pl.ds(i * jnp.int32(16), 16)
                cur = ids_v[sl]
                prv = prev_v[sl]
                p_hi = lax.shift_right_logical(prv, jnp.int32(10))
                p_lo = lax.bitwise_and(prv, jnp.int32(1023))
                h = (p_hi * c_hi + p_lo * c_lo + cur) % _TABLE
                if remap:
                    m = lax.bitwise_and(h, jnp.int32(_QP - 1))
                    a = lax.shift_right_logical(h, jnp.int32(18))
                    hp = lax.bitwise_or(lax.shift_left(m, jnp.int32(2)), a)
                else:
                    hp = h
                u0 = lax.shift_left(
                    lax.bitwise_and(i, jnp.int32(7)), jnp.int32(4))
                qa = lax.shift_right_logical(i, jnp.int32(3))
                slots = ((u0 + lax.broadcasted_iota(jnp.int32, (16,), 0))
                         * jnp.int32(4) + qa)
                plsc.store_scatter(idx_v, [slots], hp)
                return hcarry

            lax.fori_loop(jnp.int32(0), jnp.int32(_CH // 16), hstep,
                          jnp.int32(0))
            cps = [
                pltpu.async_copy(
                    t_h.at[idx_v.at[pl.ds(jnp.int32(s * _SUB), _SUB)]],
                    rows_v.at[pl.ds(jnp.int32(s * _SUB), _SUB)], sem)
                for s in range(_NSUB)
            ]
            for cp in cps:
                cp.wait()
            pltpu.sync_copy(rows_v, e_h.at[pl.ds(base, _CH)])
            return carry

        lax.fori_loop(jnp.int32(0), nch, chunk, jnp.int32(0))

    return k(ids, prev, table_pk)


def _tc_project(e1, e2, e3, w, n):
    bm = 2048
    bp = bm // 4

    def body(e1_r, e2_r, e3_r, w_r, o_r):
        for c in range(4):
            for a in range(4):
                cat = jnp.concatenate(
                    [e_r[pl.ds(c * _SUB, _SUB),
                         pl.ds(a * _D, _D)]
                     for e_r in (e1_r, e2_r, e3_r)], axis=1)
                o_r[pl.ds(c * _CH + a * _SUB, _SUB), :] = lax.dot_general(
                    cat, w_r[...], (((1,), (1,)), ((), ())),
                    preferred_element_type=jnp.float32)

    return pl.pallas_call(
        body,
        grid=(n // bm,),
        in_specs=[
            pl.BlockSpec((bp, 4 * _D), lambda i: (i, jnp.int32(0))),
            pl.BlockSpec((bp, 4 * _D), lambda i: (i, jnp.int32(0))),
            pl.BlockSpec((bp, 4 * _D), lambda i: (i, jnp.int32(0))),
            pl.BlockSpec((_H, 3 * _D), lambda i: (jnp.int32(0), jnp.int32(0))),
        ],
        out_specs=pl.BlockSpec((bm, _H), lambda i: (i, jnp.int32(0))),
        out_shape=jax.ShapeDtypeStruct((n, _H), jnp.float32),
    )(e1.reshape(n // 4, 4 * _D), e2.reshape(n // 4, 4 * _D),
      e3.reshape(n // 4, 4 * _D), w)


def kernel(input_ids, table1, table2, table3, W):
    b, t = input_ids.shape
    n = b * t
    ids32 = input_ids.astype(jnp.int32)
    prev = jnp.concatenate(
        [jnp.zeros((b, 1), jnp.int32), ids32[:, :-1]], axis=1)
    ids_f = ids32.reshape(-1)
    prev_f = prev.reshape(-1)
    es = []
    pk1 = _sc_repack(table1.T).reshape(_TABLE, _D)
    es.append(_sc_gather_one(
        ids_f, prev_f, pk1, n, *_HASH_C[0], remap=False))
    for tab, (c_hi, c_lo) in zip((table2, table3), _HASH_C[1:]):
        t_pk = _tc_transpose_packed(tab.T).reshape(4 * _QP, _D)
        es.append(_sc_gather_one(
            ids_f, prev_f, t_pk, n, c_hi, c_lo, remap=True))
    out = _tc_project(es[0], es[1], es[2], W, n)
    return out.reshape(b, t, _H)


# speedup vs baseline: 2.5194x; 2.5194x over previous
import functools

import jax
import jax.numpy as jnp
from jax import lax
from jax.experimental import pallas as pl
from jax.experimental.pallas import tpu as pltpu
from jax.experimental.pallas import tpu_sc as plsc

_TABLE = 1000000
_D = 32
_H = 128
_NC, _NS = 2, 16
_NW = _NC * _NS
_CH = 512
_SUB = 128
_NSUB = _CH // _SUB

_HASH_C = (
    (387584, 8191),
    (242496, 104729),
    (935232, 97593),
)

_QP = 262144
_TBN = 4096
_LASTB = (_TABLE - 1) // _TBN


def _tc_transpose_packed(table_t):
    def body(t0, t1, t2, t3, o_r):
        eye = (lax.broadcasted_iota(jnp.int32, (_H, _H), 0)
               == lax.broadcasted_iota(jnp.int32, (_H, _H), 1)
               ).astype(jnp.float32)
        tall = jnp.concatenate([t[...] for t in (t0, t1, t2, t3)], axis=0)
        o_r[...] = lax.dot_general(
            tall, eye, (((0,), (0,)), ((), ())),
            preferred_element_type=jnp.float32)

    def in_spec(a):
        base = a * (_QP // _TBN)
        return pl.BlockSpec(
            (_D, _TBN),
            lambda i: (jnp.int32(0),
                       jnp.minimum(jnp.int32(base) + i, jnp.int32(_LASTB))))

    return pl.pallas_call(
        body,
        grid=(_QP // _TBN,),
        in_specs=[in_spec(a) for a in range(4)],
        out_specs=pl.BlockSpec((_TBN, 4 * _D), lambda i: (i, jnp.int32(0))),
        out_shape=jax.ShapeDtypeStruct((_QP, 4 * _D), jnp.float32),
    )(table_t, table_t, table_t, table_t)


def _sc_gather_one(ids, prev, table_pk, n, c_hi, c_lo):
    nchunks = n // _CH
    base_per_w = nchunks // _NW
    extra = nchunks % _NW
    mesh = plsc.VectorSubcoreMesh(core_axis_name="c", subcore_axis_name="s")

    @functools.partial(
        pl.kernel,
        out_type=jax.ShapeDtypeStruct((n, _D), jnp.float32),
        mesh=mesh,
        scratch_types=[
            pltpu.VMEM((_CH,), jnp.int32),
            pltpu.VMEM((_CH,), jnp.int32),
            pltpu.VMEM((_CH,), jnp.int32),
            pltpu.VMEM((_CH, _D), jnp.float32),
            pltpu.SemaphoreType.DMA,
        ],
        compiler_params=pltpu.CompilerParams(
            use_tc_tiling_on_sc=False, needs_layout_passes=False),
    )
    def k(ids_h, prev_h, t_h, e_h, ids_v, prev_v, idx_v, rows_v, sem):
        wid = lax.axis_index("s") * _NC + lax.axis_index("c")
        nch = jnp.int32(base_per_w) + (wid < extra).astype(jnp.int32)

        def chunk(c, carry):
            base = (c * jnp.int32(_NW) + wid) * jnp.int32(_CH)
            pltpu.sync_copy(ids_h.at[pl.ds(base, _CH)], ids_v)
            pltpu.sync_copy(prev_h.at[pl.ds(base, _CH)], prev_v)

            def hstep(i, hcarry):
                sl = pl.ds(i * jnp.int32(16), 16)
                cur = ids_v[sl]
                prv = prev_v[sl]
                p_hi = lax.shift_right_logical(prv, jnp.int32(10))
                p_lo = lax.bitwise_and(prv, jnp.int32(1023))
                h = (p_hi * c_hi + p_lo * c_lo + cur) % _TABLE
                m = lax.bitwise_and(h, jnp.int32(_QP - 1))
                a = lax.shift_right_logical(h, jnp.int32(18))
                hp = lax.bitwise_or(lax.shift_left(m, jnp.int32(2)), a)
                u0 = lax.shift_left(
                    lax.bitwise_and(i, jnp.int32(7)), jnp.int32(4))
                qa = lax.shift_right_logical(i, jnp.int32(3))
                slots = ((u0 + lax.broadcasted_iota(jnp.int32, (16,), 0))
                         * jnp.int32(4) + qa)
                plsc.store_scatter(idx_v, [slots], hp)
                return hcarry

            lax.fori_loop(jnp.int32(0), jnp.int32(_CH // 16), hstep,
                          jnp.int32(0))
            cps = [
                pltpu.async_copy(
                    t_h.at[idx_v.at[pl.ds(jnp.int32(s * _SUB), _SUB)]],
                    rows_v.at[pl.ds(jnp.int32(s * _SUB), _SUB)], sem)
                for s in range(_NSUB)
            ]
            for cp in cps:
                cp.wait()
            pltpu.sync_copy(rows_v, e_h.at[pl.ds(base, _CH)])
            return carry

        lax.fori_loop(jnp.int32(0), nch, chunk, jnp.int32(0))

    return k(ids, prev, table_pk)


def _tc_project(e1, e2, e3, w, n):
    bm = 2048
    bp = bm // 4

    def body(e1_r, e2_r, e3_r, w_r, o_r):
        for c in range(4):
            for a in range(4):
                cat = jnp.concatenate(
                    [e_r[pl.ds(c * _SUB, _SUB),
                         pl.ds(a * _D, _D)]
                     for e_r in (e1_r, e2_r, e3_r)], axis=1)
                o_r[pl.ds(c * _CH + a * _SUB, _SUB), :] = lax.dot_general(
                    cat, w_r[...], (((1,), (1,)), ((), ())),
                    preferred_element_type=jnp.float32)

    return pl.pallas_call(
        body,
        grid=(n // bm,),
        in_specs=[
            pl.BlockSpec((bp, 4 * _D), lambda i: (i, jnp.int32(0))),
            pl.BlockSpec((bp, 4 * _D), lambda i: (i, jnp.int32(0))),
            pl.BlockSpec((bp, 4 * _D), lambda i: (i, jnp.int32(0))),
            pl.BlockSpec((_H, 3 * _D), lambda i: (jnp.int32(0), jnp.int32(0))),
        ],
        out_specs=pl.BlockSpec((bm, _H), lambda i: (i, jnp.int32(0))),
        out_shape=jax.ShapeDtypeStruct((n, _H), jnp.float32),
    )(e1.reshape(n // 4, 4 * _D), e2.reshape(n // 4, 4 * _D),
      e3.reshape(n // 4, 4 * _D), w)


def kernel(input_ids, table1, table2, table3, W):
    b, t = input_ids.shape
    n = b * t
    ids32 = input_ids.astype(jnp.int32)
    prev = jnp.concatenate(
        [jnp.zeros((b, 1), jnp.int32), ids32[:, :-1]], axis=1)
    ids_f = ids32.reshape(-1)
    prev_f = prev.reshape(-1)
    es = []
    for tab, (c_hi, c_lo) in zip((table1, table2, table3), _HASH_C):
        t_pk = _tc_transpose_packed(tab.T).reshape(4 * _QP, _D)
        es.append(_sc_gather_one(ids_f, prev_f, t_pk, n, c_hi, c_lo))
    out = _tc_project(es[0], es[1], es[2], W, n)
    return out.reshape(b, t, _H)


# speedup vs baseline: 2.6773x; 1.0627x over previous
import functools

import jax
import jax.numpy as jnp
from jax import lax
from jax.experimental import pallas as pl
from jax.experimental.pallas import tpu as pltpu
from jax.experimental.pallas import tpu_sc as plsc

_TABLE = 1000000
_D = 32
_H = 128
_NC, _NS = 2, 16
_NW = _NC * _NS
_CH = 512
_SUB = 128
_NSUB = _CH // _SUB

_HASH_C = (
    (387584, 8191),
    (242496, 104729),
    (935232, 97593),
)

_QP = 262144
_TBN = 8192
_LASTB = (_TABLE - 1) // _TBN


def _tc_transpose_packed(table_t):
    def body(t0, t1, t2, t3, o_r):
        eye = (lax.broadcasted_iota(jnp.int32, (_H, _H), 0)
               == lax.broadcasted_iota(jnp.int32, (_H, _H), 1)
               ).astype(jnp.float32)
        tall = jnp.concatenate([t[...] for t in (t0, t1, t2, t3)], axis=0)
        o_r[...] = lax.dot_general(
            tall, eye, (((0,), (0,)), ((), ())),
            preferred_element_type=jnp.float32)

    def in_spec(a):
        base = a * (_QP // _TBN)
        return pl.BlockSpec(
            (_D, _TBN),
            lambda i: (jnp.int32(0),
                       jnp.minimum(jnp.int32(base) + i, jnp.int32(_LASTB))))

    return pl.pallas_call(
        body,
        grid=(_QP // _TBN,),
        in_specs=[in_spec(a) for a in range(4)],
        out_specs=pl.BlockSpec((_TBN, 4 * _D), lambda i: (i, jnp.int32(0))),
        out_shape=jax.ShapeDtypeStruct((_QP, 4 * _D), jnp.float32),
    )(table_t, table_t, table_t, table_t)


def _sc_gather_one(ids, prev, table_pk, n, c_hi, c_lo):
    nchunks = n // _CH
    base_per_w = nchunks // _NW
    extra = nchunks % _NW
    mesh = plsc.VectorSubcoreMesh(core_axis_name="c", subcore_axis_name="s")

    @functools.partial(
        pl.kernel,
        out_type=jax.ShapeDtypeStruct((n, _D), jnp.float32),
        mesh=mesh,
        scratch_types=[
            pltpu.VMEM((_CH,), jnp.int32),
            pltpu.VMEM((_CH,), jnp.int32),
            pltpu.VMEM((_CH,), jnp.int32),
            pltpu.VMEM((_CH, _D), jnp.float32),
            pltpu.SemaphoreType.DMA,
        ],
        compiler_params=pltpu.CompilerParams(
            use_tc_tiling_on_sc=False, needs_layout_passes=False),
    )
    def k(ids_h, prev_h, t_h, e_h, ids_v, prev_v, idx_v, rows_v, sem):
        wid = lax.axis_index("s") * _NC + lax.axis_index("c")
        nch = jnp.int32(base_per_w) + (wid < extra).astype(jnp.int32)

        def chunk(c, carry):
            base = (c * jnp.int32(_NW) + wid) * jnp.int32(_CH)
            pltpu.sync_copy(ids_h.at[pl.ds(base, _CH)], ids_v)
            pltpu.sync_copy(prev_h.at[pl.ds(base, _CH)], prev_v)

            def hstep(i, hcarry):
                sl = pl.ds(i * jnp.int32(16), 16)
                cur = ids_v[sl]
                prv = prev_v[sl]
                p_hi = lax.shift_right_logical(prv, jnp.int32(10))
                p_lo = lax.bitwise_and(prv, jnp.int32(1023))
                h = (p_hi * c_hi + p_lo * c_lo + cur) % _TABLE
                m = lax.bitwise_and(h, jnp.int32(_QP - 1))
                a = lax.shift_right_logical(h, jnp.int32(18))
                hp = lax.bitwise_or(lax.shift_left(m, jnp.int32(2)), a)
                u0 = lax.shift_left(
                    lax.bitwise_and(i, jnp.int32(7)), jnp.int32(4))
                qa = lax.shift_right_logical(i, jnp.int32(3))
                slots = ((u0 + lax.broadcasted_iota(jnp.int32, (16,), 0))
                         * jnp.int32(4) + qa)
                plsc.store_scatter(idx_v, [slots], hp)
                return hcarry

            lax.fori_loop(jnp.int32(0), jnp.int32(_CH // 16), hstep,
                          jnp.int32(0))
            cps = [
                pltpu.async_copy(
                    t_h.at[idx_v.at[pl.ds(jnp.int32(s * _SUB), _SUB)]],
                    rows_v.at[pl.ds(jnp.int32(s * _SUB), _SUB)], sem)
                for s in range(_NSUB)
            ]
            for cp in cps:
                cp.wait()
            pltpu.sync_copy(rows_v, e_h.at[pl.ds(base, _CH)])
            return carry

        lax.fori_loop(jnp.int32(0), nch, chunk, jnp.int32(0))

    return k(ids, prev, table_pk)


def _tc_project(e1, e2, e3, w, n):
    bm = 2048
    bp = bm // 4

    def body(e1_r, e2_r, e3_r, w_r, o_r):
        for c in range(4):
            for a in range(4):
                cat = jnp.concatenate(
                    [e_r[pl.ds(c * _SUB, _SUB),
                         pl.ds(a * _D, _D)]
                     for e_r in (e1_r, e2_r, e3_r)], axis=1)
                o_r[pl.ds(c * _CH + a * _SUB, _SUB), :] = lax.dot_general(
                    cat, w_r[...], (((1,), (1,)), ((), ())),
                    preferred_element_type=jnp.float32)

    return pl.pallas_call(
        body,
        grid=(n // bm,),
        in_specs=[
            pl.BlockSpec((bp, 4 * _D), lambda i: (i, jnp.int32(0))),
            pl.BlockSpec((bp, 4 * _D), lambda i: (i, jnp.int32(0))),
            pl.BlockSpec((bp, 4 * _D), lambda i: (i, jnp.int32(0))),
            pl.BlockSpec((_H, 3 * _D), lambda i: (jnp.int32(0), jnp.int32(0))),
        ],
        out_specs=pl.BlockSpec((bm, _H), lambda i: (i, jnp.int32(0))),
        out_shape=jax.ShapeDtypeStruct((n, _H), jnp.float32),
    )(e1.reshape(n // 4, 4 * _D), e2.reshape(n // 4, 4 * _D),
      e3.reshape(n // 4, 4 * _D), w)


def kernel(input_ids, table1, table2, table3, W):
    b, t = input_ids.shape
    n = b * t
    ids32 = input_ids.astype(jnp.int32)
    prev = jnp.concatenate(
        [jnp.zeros((b, 1), jnp.int32), ids32[:, :-1]], axis=1)
    ids_f = ids32.reshape(-1)
    prev_f = prev.reshape(-1)
    es = []
    for tab, (c_hi, c_lo) in zip((table1, table2, table3), _HASH_C):
        t_pk = _tc_transpose_packed(tab.T).reshape(4 * _QP, _D)
        es.append(_sc_gather_one(ids_f, prev_f, t_pk, n, c_hi, c_lo))
    out = _tc_project(es[0], es[1], es[2], W, n)
    return out.reshape(b, t, _H)
